# baseline (speedup 1.0000x reference)
import functools

import jax
import jax.numpy as jnp
from jax import lax
from jax.experimental import pallas as pl
from jax.experimental.pallas import tpu as pltpu
from jax.experimental.pallas import tpu_sc as plsc

_BATCH = 16384
_N_NUM = 13
_N_EMB = 26
_EMB_DIM = 16
_N_OH = 13
_OH_CARD = 8

_NC = 2
_NS = 16
_NW = _NC * _NS
_BPW = _BATCH // _NW
_CHUNKS = _BPW // 16

_DROWS = 32
_EMB_OFF = _N_NUM
_OH_OFF = _N_NUM + _N_EMB * _EMB_DIM
_WNUM_OFF = 544
_FCB_OFF = 752
_WPAD = 768


def _full(val):
    return jnp.full((16,), val, jnp.int32)


def _sc_body(num_hbm, idx_hbm, oh_hbm, tab_hbm, wv_hbm, out_hbm,
             num_v, idx_v, oh_v, tabs_v, wv_v, d_v, out_v, dma_sem):
    wid = lax.axis_index("s") * _NC + lax.axis_index("c")
    base = wid * _BPW

    copies = [
        pltpu.async_copy(wv_hbm, wv_v, dma_sem),
        pltpu.async_copy(num_hbm.at[pl.ds(base, _BPW), :], num_v, dma_sem),
        pltpu.async_copy(idx_hbm.at[pl.ds(base, _BPW), :], idx_v, dma_sem),
        pltpu.async_copy(oh_hbm.at[pl.ds(base, _BPW), :], oh_v, dma_sem),
        pltpu.async_copy(tab_hbm, tabs_v, dma_sem),
    ]
    for cp in copies:
        cp.wait()

    viota = lax.broadcasted_iota(jnp.int32, (16,), 0)

    def d_field(i, carry):
        acc0 = jnp.zeros((16,), jnp.float32)
        acc1 = jnp.zeros((16,), jnp.float32)
        vrow = viota * _EMB_DIM
        for d in range(_EMB_DIM):
            w_sd = plsc.load_gather(wv_v, [_full(_EMB_OFF + d) + i * _EMB_DIM])
            fbase = i * (_DROWS * _EMB_DIM) + d
            t0 = plsc.load_gather(tabs_v, [vrow + fbase])
            t1 = plsc.load_gather(tabs_v, [vrow + (fbase + 16 * _EMB_DIM)])
            acc0 = acc0 + t0 * w_sd
            acc1 = acc1 + t1 * w_sd
        d_v[pl.ds(i * _DROWS, 16)] = acc0
        d_v[pl.ds(i * _DROWS + 16, 16)] = acc1
        return carry

    lax.fori_loop(0, _N_EMB, d_field, 0)

    fcb = wv_v[pl.ds(_FCB_OFF, 16)]
    wnum = [wv_v[pl.ds(_WNUM_OFF + 16 * j, 16)] for j in range(_N_NUM)]

    def chunk(c, carry):
        rows = c * 16 + viota
        acc = fcb
        for j in range(_N_NUM):
            v = plsc.load_gather(num_v, [rows, _full(j)])
            acc = acc + v * wnum[j]
        for i in range(_N_EMB):
            iv = plsc.load_gather(idx_v, [rows, _full(i)])
            iv = jnp.clip(iv, 0, _N_EMB - 1)
            acc = acc + plsc.load_gather(d_v, [iv + i * _DROWS])
        for j in range(_N_OH):
            ov = plsc.load_gather(oh_v, [rows, _full(j)])
            ov = jnp.clip(ov, 0, _OH_CARD - 1)
            acc = acc + plsc.load_gather(wv_v, [ov + (_OH_OFF + j * _OH_CARD)])
        out_v[pl.ds(c * 16, 16)] = acc
        return carry

    lax.fori_loop(0, _CHUNKS, chunk, 0)

    pltpu.sync_copy(out_v, out_hbm.at[pl.ds(base, _BPW)])


_sc_forward = functools.partial(
    pl.kernel,
    mesh=plsc.VectorSubcoreMesh(core_axis_name="c", subcore_axis_name="s"),
    out_type=jax.ShapeDtypeStruct((_BATCH,), jnp.float32),
    compiler_params=pltpu.CompilerParams(
        needs_layout_passes=False, use_tc_tiling_on_sc=False),
    scratch_types=[
        pltpu.VMEM((_BPW, _N_NUM), jnp.float32),
        pltpu.VMEM((_BPW, _N_EMB), jnp.int32),
        pltpu.VMEM((_BPW, _N_OH), jnp.int32),
        pltpu.VMEM((_N_EMB * _DROWS * _EMB_DIM,), jnp.float32),
        pltpu.VMEM((_WPAD,), jnp.float32),
        pltpu.VMEM((_N_EMB * _DROWS,), jnp.float32),
        pltpu.VMEM((_BPW,), jnp.float32),
        pltpu.SemaphoreType.DMA,
    ],
)(_sc_body)


def kernel(num_features, cat_emb_features, cat_one_hot_features, emb_tables, fc_w, fc_b):
    idx = cat_emb_features.astype(jnp.int32)
    oh = cat_one_hot_features.astype(jnp.int32)
    wv = jnp.concatenate([
        fc_w[:, 0],
        jnp.zeros((_WNUM_OFF - 533,), jnp.float32),
        jnp.repeat(fc_w[:_N_NUM, 0], 16),
        jnp.broadcast_to(fc_b, (16,)),
    ])
    tabs = emb_tables[:, :_DROWS, :].reshape(-1)
    out = _sc_forward(num_features, idx, oh, tabs, wv)
    return out.reshape(_BATCH, 1)


# speedup vs baseline: 1.2137x; 1.2137x over previous
import functools

import jax
import jax.numpy as jnp
from jax import lax
from jax.experimental import pallas as pl
from jax.experimental.pallas import tpu as pltpu
from jax.experimental.pallas import tpu_sc as plsc

_BATCH = 16384
_N_NUM = 13
_N_EMB = 26
_EMB_DIM = 16
_N_OH = 13
_OH_CARD = 8

_NC = 2
_NS = 16
_NW = _NC * _NS
_BPW = _BATCH // _NW
_CHUNKS = _BPW // 16

_DROWS = 32
_EMB_OFF = _N_NUM
_OH_OFF = _N_NUM + _N_EMB * _EMB_DIM
_WNUM_OFF = 544
_FCB_OFF = 752
_WPAD = 768


def _full(val):
    return jnp.full((16,), val, jnp.int32)


def _sc_body(num_hbm, idx_hbm, oh_hbm, tab_hbm, wv_hbm, out_hbm,
             num_v, idx_v, oh_v, tabs_v, wv_v, d_v, out_v, dma_sem):
    wid = lax.axis_index("s") * _NC + lax.axis_index("c")
    base = wid * _BPW

    copies = [
        pltpu.async_copy(wv_hbm, wv_v, dma_sem),
    ] + [
        pltpu.async_copy(num_hbm.at[pl.ds(j * _BATCH + base, _BPW)],
                         num_v.at[pl.ds(j * _BPW, _BPW)], dma_sem)
        for j in range(_N_NUM)
    ] + [
        pltpu.async_copy(idx_hbm.at[pl.ds(base, _BPW), :], idx_v, dma_sem),
        pltpu.async_copy(oh_hbm.at[pl.ds(base, _BPW), :], oh_v, dma_sem),
        pltpu.async_copy(tab_hbm, tabs_v, dma_sem),
    ]
    for cp in copies:
        cp.wait()

    viota = lax.broadcasted_iota(jnp.int32, (16,), 0)

    def d_field(i, carry):
        acc0 = jnp.zeros((16,), jnp.float32)
        acc1 = jnp.zeros((16,), jnp.float32)
        vrow = viota * _EMB_DIM
        for d in range(_EMB_DIM):
            w_sd = plsc.load_gather(wv_v, [_full(_EMB_OFF + d) + i * _EMB_DIM])
            fbase = i * (_DROWS * _EMB_DIM) + d
            t0 = plsc.load_gather(tabs_v, [vrow + fbase])
            t1 = plsc.load_gather(tabs_v, [vrow + (fbase + 16 * _EMB_DIM)])
            acc0 = acc0 + t0 * w_sd
            acc1 = acc1 + t1 * w_sd
        d_v[pl.ds(i * _DROWS, 16)] = acc0
        d_v[pl.ds(i * _DROWS + 16, 16)] = acc1
        return carry

    lax.fori_loop(0, _N_EMB, d_field, 0)

    fcb = wv_v[pl.ds(_FCB_OFF, 16)]
    wnum = [wv_v[pl.ds(_WNUM_OFF + 16 * j, 16)] for j in range(_N_NUM)]

    def chunk(c, carry):
        rows = c * 16 + viota
        acc = fcb
        for j in range(_N_NUM):
            v = num_v[pl.ds(j * _BPW + c * 16, 16)]
            acc = acc + v * wnum[j]
        for i in range(_N_EMB):
            iv = plsc.load_gather(idx_v, [rows, _full(i)])
            iv = jnp.clip(iv, 0, _N_EMB - 1)
            acc = acc + plsc.load_gather(d_v, [iv + i * _DROWS])
        for j in range(_N_OH):
            ov = plsc.load_gather(oh_v, [rows, _full(j)])
            ov = jnp.clip(ov, 0, _OH_CARD - 1)
            acc = acc + plsc.load_gather(wv_v, [ov + (_OH_OFF + j * _OH_CARD)])
        out_v[pl.ds(c * 16, 16)] = acc
        return carry

    lax.fori_loop(0, _CHUNKS, chunk, 0)

    pltpu.sync_copy(out_v, out_hbm.at[pl.ds(base, _BPW)])


_sc_forward = functools.partial(
    pl.kernel,
    mesh=plsc.VectorSubcoreMesh(core_axis_name="c", subcore_axis_name="s"),
    out_type=jax.ShapeDtypeStruct((_BATCH,), jnp.float32),
    compiler_params=pltpu.CompilerParams(
        needs_layout_passes=False, use_tc_tiling_on_sc=False),
    scratch_types=[
        pltpu.VMEM((_BPW * _N_NUM,), jnp.float32),
        pltpu.VMEM((_BPW, _N_EMB), jnp.int32),
        pltpu.VMEM((_BPW, _N_OH), jnp.int32),
        pltpu.VMEM((_N_EMB * _DROWS * _EMB_DIM,), jnp.float32),
        pltpu.VMEM((_WPAD,), jnp.float32),
        pltpu.VMEM((_N_EMB * _DROWS,), jnp.float32),
        pltpu.VMEM((_BPW,), jnp.float32),
        pltpu.SemaphoreType.DMA,
    ],
)(_sc_body)


def kernel(num_features, cat_emb_features, cat_one_hot_features, emb_tables, fc_w, fc_b):
    idx = cat_emb_features.astype(jnp.int32)
    oh = cat_one_hot_features.astype(jnp.int32)
    wv = jnp.concatenate([
        fc_w[:, 0],
        jnp.zeros((_WNUM_OFF - 533,), jnp.float32),
        jnp.repeat(fc_w[:_N_NUM, 0], 16),
        jnp.broadcast_to(fc_b, (16,)),
    ])
    tabs = emb_tables[:, :_DROWS, :].reshape(-1)
    out = _sc_forward(num_features.T.reshape(-1), idx, oh, tabs, wv)
    return out.reshape(_BATCH, 1)


# speedup vs baseline: 1.2154x; 1.0014x over previous
import functools

import jax
import jax.numpy as jnp
from jax import lax
from jax.experimental import pallas as pl
from jax.experimental.pallas import tpu as pltpu
from jax.experimental.pallas import tpu_sc as plsc

_BATCH = 16384
_N_NUM = 13
_N_EMB = 26
_EMB_DIM = 16
_N_OH = 13
_OH_CARD = 8

_NC = 2
_NS = 16
_NW = _NC * _NS
_BPW = _BATCH // _NW
_CHUNKS = _BPW // 16

_DROWS = 32
_EMB_OFF = _N_NUM
_OH_OFF = _N_NUM + _N_EMB * _EMB_DIM
_WNUM_OFF = 544
_FCB_OFF = 752
_WPAD = 768


def _full(val):
    return jnp.full((16,), val, jnp.int32)


def _sc_body(num_hbm, idx_hbm, oh_hbm, tab_hbm, wv_hbm, out_hbm,
             num_v, idx_v, oh_v, tabs_v, wv_v, d_v, out_v, dma_sem):
    wid = lax.axis_index("s") * _NC + lax.axis_index("c")
    base = wid * _BPW

    copies = [
        pltpu.async_copy(wv_hbm, wv_v, dma_sem),
    ] + [
        pltpu.async_copy(num_hbm.at[pl.ds(j * _BATCH + base, _BPW)],
                         num_v.at[pl.ds(j * _BPW, _BPW)], dma_sem)
        for j in range(_N_NUM)
    ] + [
        pltpu.async_copy(idx_hbm.at[pl.ds(base, _BPW), :], idx_v, dma_sem),
        pltpu.async_copy(oh_hbm.at[pl.ds(base, _BPW), :], oh_v, dma_sem),
        pltpu.async_copy(tab_hbm, tabs_v, dma_sem),
    ]
    for cp in copies:
        cp.wait()

    viota = lax.broadcasted_iota(jnp.int32, (16,), 0)

    def d_field(i, carry):
        acc0 = jnp.zeros((16,), jnp.float32)
        acc1 = jnp.zeros((16,), jnp.float32)
        vrow = viota * _EMB_DIM
        for d in range(_EMB_DIM):
            w_sd = plsc.load_gather(wv_v, [_full(_EMB_OFF + d) + i * _EMB_DIM])
            fbase = i * (_DROWS * _EMB_DIM) + d
            t0 = plsc.load_gather(tabs_v, [vrow + fbase])
            t1 = plsc.load_gather(tabs_v, [vrow + (fbase + 16 * _EMB_DIM)])
            acc0 = acc0 + t0 * w_sd
            acc1 = acc1 + t1 * w_sd
        d_v[pl.ds(i * _DROWS, 16)] = acc0
        d_v[pl.ds(i * _DROWS + 16, 16)] = acc1
        return carry

    lax.fori_loop(0, _N_EMB, d_field, 0)

    fcb = wv_v[pl.ds(_FCB_OFF, 16)]
    wnum = [wv_v[pl.ds(_WNUM_OFF + 16 * j, 16)] for j in range(_N_NUM)]

    def chunk(c, carry):
        rows = c * 16 + viota
        acc = fcb
        for j in range(_N_NUM):
            v = num_v[pl.ds(j * _BPW + c * 16, 16)]
            acc = acc + v * wnum[j]
        for i in range(_N_EMB):
            iv = plsc.load_gather(idx_v, [rows, _full(i)])
            iv = jnp.clip(iv, 0, _N_EMB - 1)
            acc = acc + plsc.load_gather(d_v, [iv + i * _DROWS])
        for j in range(_N_OH):
            ov = plsc.load_gather(oh_v, [rows, _full(j)])
            ov = jnp.clip(ov, 0, _OH_CARD - 1)
            acc = acc + plsc.load_gather(wv_v, [ov + (_OH_OFF + j * _OH_CARD)])
        out_v[pl.ds(c * 16, 16)] = acc
        return carry

    lax.fori_loop(0, _CHUNKS, chunk, 0)

    pltpu.sync_copy(out_v, out_hbm.at[pl.ds(base, _BPW)])


_sc_forward = functools.partial(
    pl.kernel,
    mesh=plsc.VectorSubcoreMesh(core_axis_name="c", subcore_axis_name="s"),
    out_type=jax.ShapeDtypeStruct((_BATCH,), jnp.float32),
    compiler_params=pltpu.CompilerParams(
        needs_layout_passes=False, use_tc_tiling_on_sc=False,
        disable_bounds_checks=True),
    scratch_types=[
        pltpu.VMEM((_BPW * _N_NUM,), jnp.float32),
        pltpu.VMEM((_BPW, _N_EMB), jnp.int32),
        pltpu.VMEM((_BPW, _N_OH), jnp.int32),
        pltpu.VMEM((_N_EMB * _DROWS * _EMB_DIM,), jnp.float32),
        pltpu.VMEM((_WPAD,), jnp.float32),
        pltpu.VMEM((_N_EMB * _DROWS,), jnp.float32),
        pltpu.VMEM((_BPW,), jnp.float32),
        pltpu.SemaphoreType.DMA,
    ],
)(_sc_body)


def kernel(num_features, cat_emb_features, cat_one_hot_features, emb_tables, fc_w, fc_b):
    idx = cat_emb_features.astype(jnp.int32)
    oh = cat_one_hot_features.astype(jnp.int32)
    wv = jnp.concatenate([
        fc_w[:, 0],
        jnp.zeros((_WNUM_OFF - 533,), jnp.float32),
        jnp.repeat(fc_w[:_N_NUM, 0], 16),
        jnp.broadcast_to(fc_b, (16,)),
    ])
    tabs = emb_tables[:, :_DROWS, :].reshape(-1)
    out = _sc_forward(num_features.T.reshape(-1), idx, oh, tabs, wv)
    return out.reshape(_BATCH, 1)
